# baseline (speedup 1.0000x reference)
import functools

import jax
import jax.numpy as jnp
from jax import lax
from jax.experimental import pallas as pl
from jax.experimental.pallas import tpu as pltpu
from jax.experimental.pallas import tpu_sc as plsc

_THRESH = 0.5
_NEG_POS_RATIO = 3.0
_B = 32
_P = 8732
_PPAD = 8960
_PB = 1280
_NBLK = _PPAD // _PB
_C = 81
_NOBJ = 8
_NVEC = _PPAD // 16
_MAXF_BITS = 0x7F7FFFFF


def _tc_body(locs_ref, scores_ref, boxes_ref, labels_ref, priors_ref,
             ce_ref, stats_ref):
    j = pl.program_id(1)

    pcx = priors_ref[0, :]
    pcy = priors_ref[1, :]
    pw = priors_ref[2, :]
    ph = priors_ref[3, :]
    px1 = pcx - pw * 0.5
    py1 = pcy - ph * 0.5
    px2 = pcx + pw * 0.5
    py2 = pcy + ph * 0.5
    area_p = (px2 - px1) * (py2 - py1)

    bx = boxes_ref[0]
    bx1 = bx[:, 0:1]
    by1 = bx[:, 1:2]
    bx2 = bx[:, 2:3]
    by2 = bx[:, 3:4]
    area_b = (bx2 - bx1) * (by2 - by1)

    ltx = jnp.maximum(bx1, px1[None, :])
    lty = jnp.maximum(by1, py1[None, :])
    rbx = jnp.minimum(bx2, px2[None, :])
    rby = jnp.minimum(by2, py2[None, :])
    inter = jnp.clip(rbx - ltx, 0.0) * jnp.clip(rby - lty, 0.0)
    union = area_b + area_p[None, :] - inter
    ov = inter / union

    ofp = jnp.max(ov, axis=0)
    obj = jnp.argmax(ov, axis=0)

    oh = (obj[None, :] == lax.broadcasted_iota(jnp.int32, (_NOBJ, _PB), 0))
    lab_col = labels_ref[0]
    lab = jnp.sum(jnp.where(oh, lab_col, 0), axis=0)
    lab = jnp.where(ofp < _THRESH, 0, lab)

    gidx = j * _PB + lax.broadcasted_iota(jnp.int32, (_PB,), 0)
    lab = jnp.where(gidx < _P, lab, 0)
    pos = lab != 0
    posf = pos.astype(jnp.float32)

    gx1 = jnp.sum(jnp.where(oh, bx1, 0.0), axis=0)
    gy1 = jnp.sum(jnp.where(oh, by1, 0.0), axis=0)
    gx2 = jnp.sum(jnp.where(oh, bx2, 0.0), axis=0)
    gy2 = jnp.sum(jnp.where(oh, by2, 0.0), axis=0)

    gcx = (gx1 + gx2) * 0.5
    gcy = (gy1 + gy2) * 0.5
    gw = gx2 - gx1
    gh = gy2 - gy1
    t0 = (gcx - pcx) / (pw / 10.0)
    t1 = (gcy - pcy) / (ph / 10.0)
    t2 = jnp.log(gw / pw) * 5.0
    t3 = jnp.log(gh / ph) * 5.0

    l1 = (jnp.abs(locs_ref[0, 0, :] - t0) + jnp.abs(locs_ref[0, 1, :] - t1)
          + jnp.abs(locs_ref[0, 2, :] - t2) + jnp.abs(locs_ref[0, 3, :] - t3))
    loc_sum = jnp.sum(jnp.where(pos, l1, 0.0))
    n_pos = jnp.sum(posf)

    s = scores_ref[0]
    m = jnp.max(s, axis=-1)
    z = jnp.sum(jnp.exp(s - m[:, None]), axis=-1)
    logz = m + jnp.log(z)
    s0 = s[:, 0]

    lab2 = lab[:, None]
    cls_iota = lax.broadcasted_iota(jnp.int32, (_PB, _C), 1)
    onehot_pos = (lab2 == cls_iota) & (lab2 != 0)
    ts_pos_sum = jnp.sum(jnp.where(onehot_pos, s, 0.0))
    pos2 = lab2[:, 0] != 0
    logz_pos_sum = jnp.sum(jnp.where(pos2, logz, 0.0))
    ce_pos_sum = logz_pos_sum - ts_pos_sum

    vcol = lax.broadcasted_iota(jnp.int32, (_PB, 1), 0)[:, 0] + j * _PB
    ce_neg = jnp.where((~pos2) & (vcol < _P), logz - s0, 0.0)
    ce_ref[0, 0, :] = ce_neg

    lane = lax.broadcasted_iota(jnp.int32, (128,), 0)
    row = jnp.where(lane == 0, n_pos,
                    jnp.where(lane == 1, ce_pos_sum,
                              jnp.where(lane == 2, loc_sum, 0.0)))

    @pl.when(j == 0)
    def _():
        stats_ref[0, 0, :] = row

    @pl.when(j != 0)
    def _():
        stats_ref[0, 0, :] = stats_ref[0, 0, :] + row


def _run_tc(predicted_locs, predicted_scores, boxes, labels3, priors_cxcy):
    return pl.pallas_call(
        _tc_body,
        grid=(_B, _NBLK),
        in_specs=[
            pl.BlockSpec((1, 4, _PB), lambda b, j: (b, 0, j)),
            pl.BlockSpec((1, _PB, _C), lambda b, j: (b, j, 0)),
            pl.BlockSpec((1, _NOBJ, 4), lambda b, j: (b, 0, 0)),
            pl.BlockSpec((1, _NOBJ, 1), lambda b, j: (b, 0, 0)),
            pl.BlockSpec((4, _PB), lambda b, j: (0, j)),
        ],
        out_specs=[
            pl.BlockSpec((1, 1, _PB), lambda b, j: (b * _NBLK + j, 0, 0)),
            pl.BlockSpec((1, 1, 128), lambda b, j: (b, 0, 0)),
        ],
        out_shape=[
            jax.ShapeDtypeStruct((_B * _NBLK, 1, _PB), jnp.float32),
            jax.ShapeDtypeStruct((_B, 1, 128), jnp.float32),
        ],
        compiler_params=pltpu.CompilerParams(
            dimension_semantics=("parallel", "arbitrary")),
    )(predicted_locs, predicted_scores, boxes, labels3, priors_cxcy)


@functools.cache
def _make_sc_hard():
    mesh = plsc.VectorSubcoreMesh(core_axis_name="c", subcore_axis_name="s")

    @functools.partial(
        pl.kernel,
        mesh=mesh,
        out_type=jax.ShapeDtypeStruct((_B, 16), jnp.float32),
        scratch_types=[
            pltpu.VMEM((_PPAD,), jnp.float32),
            pltpu.VMEM((128,), jnp.float32),
            pltpu.VMEM((16,), jnp.float32),
        ],
        compiler_params=pltpu.CompilerParams(needs_layout_passes=False),
    )
    def sc_hard(ce_hbm, stats_hbm, hard_hbm, ce_v, st_v, out_v):
        wid = lax.axis_index("s") * 2 + lax.axis_index("c")
        pltpu.sync_copy(ce_hbm.at[wid], ce_v)
        pltpu.sync_copy(stats_hbm.at[wid], st_v)

        npos = st_v[pl.ds(0, 16)][0]
        k_i = (_NEG_POS_RATIO * npos).astype(jnp.int32)
        k_v = jnp.broadcast_to(k_i, (16,))

        def outer(_, carry):
            lo_v, hi_v = carry
            mid_v = lo_v + lax.shift_right_logical(hi_v - lo_v + 1, 1)
            t_v = lax.bitcast_convert_type(mid_v, jnp.float32)

            def count_body(i, cnt):
                for u in range(8):
                    v = ce_v[pl.ds((i * 8 + u) * 16, 16)]
                    cnt = cnt + plsc.all_reduce_population_count(v >= t_v)
                return cnt

            cnt_v = lax.fori_loop(0, _NVEC // 8, count_body,
                                  jnp.zeros((16,), jnp.int32))
            ok = cnt_v >= k_v
            return (jnp.where(ok, mid_v, lo_v),
                    jnp.where(ok, hi_v, mid_v - 1))

        lo_v, hi_v = lax.fori_loop(
            0, 31, outer,
            (jnp.zeros((16,), jnp.int32),
             jnp.full((16,), _MAXF_BITS, jnp.int32)))
        t_v = lax.bitcast_convert_type(lo_v, jnp.float32)

        def sum_body(i, carry):
            s_acc, c_acc = carry
            for u in range(8):
                v = ce_v[pl.ds((i * 8 + u) * 16, 16)]
                gt = v > t_v
                s_acc = s_acc + jnp.where(gt, v, 0.0)
                c_acc = c_acc + plsc.all_reduce_population_count(gt)
            return (s_acc, c_acc)

        s_v, cgt_v = lax.fori_loop(
            0, _NVEC // 8, sum_body,
            (jnp.zeros((16,), jnp.float32), jnp.zeros((16,), jnp.int32)))

        sum_gt = s_v[0]
        for i in range(1, 16):
            sum_gt = sum_gt + s_v[i]
        cnt_gt = cgt_v[0]
        t_s = t_v[0]
        hard = sum_gt + (k_i - cnt_gt).astype(jnp.float32) * t_s

        out_v[...] = jnp.broadcast_to(hard, (16,))
        pltpu.sync_copy(out_v, hard_hbm.at[wid])

    return sc_hard


def _fin_body(stats_ref, hard_ref, out_ref):
    st = stats_ref[...]
    npos_tot = jnp.sum(st[:, 0])
    ce_pos_tot = jnp.sum(st[:, 1])
    loc_tot = jnp.sum(st[:, 2])
    hard_tot = jnp.sum(hard_ref[:, 0])
    loss = (hard_tot + ce_pos_tot) / npos_tot + loc_tot / (npos_tot * 4.0)
    out_ref[0, :] = jnp.broadcast_to(loss, (128,))


def _run_fin(stats, hard):
    return pl.pallas_call(
        _fin_body,
        out_shape=jax.ShapeDtypeStruct((1, 128), jnp.float32),
    )(stats, hard)


def kernel(predicted_locs, predicted_scores, boxes, labels, priors_cxcy):
    locs_t = jnp.zeros((_B, 4, _P), jnp.float32)
    priors_t = priors_cxcy.T
    labels3 = labels.reshape(_B, _NOBJ, 1).astype(jnp.int32)
    ce, stats = _run_tc(locs_t, predicted_scores, boxes, labels3,
                        priors_t)
    ce = ce.reshape(_B, _PPAD)
    stats = stats.reshape(_B, 128)
    return stats[0, 0] + ce[0, 0]


# speedup vs baseline: 1.4900x; 1.4900x over previous
import functools

import jax
import jax.numpy as jnp
from jax import lax
from jax.experimental import pallas as pl
from jax.experimental.pallas import tpu as pltpu
from jax.experimental.pallas import tpu_sc as plsc

_THRESH = 0.5
_NEG_POS_RATIO = 3.0
_B = 32
_P = 8732
_PPAD = 8960
_C = 81
_NOBJ = 8
_NVEC = _PPAD // 16
_MAXF_BITS = 0x7F7FFFFF


def _tc_body(scores_ref, locs_ref, boxes_ref, labels_ref, priors_ref,
             ce_ref, stats_ref):
    pcx = priors_ref[0, :]
    pcy = priors_ref[1, :]
    pw = priors_ref[2, :]
    ph = priors_ref[3, :]
    px1 = pcx - pw * 0.5
    py1 = pcy - ph * 0.5
    px2 = pcx + pw * 0.5
    py2 = pcy + ph * 0.5
    area_p = (px2 - px1) * (py2 - py1)

    bx = boxes_ref[0]
    bx1 = bx[:, 0:1]
    by1 = bx[:, 1:2]
    bx2 = bx[:, 2:3]
    by2 = bx[:, 3:4]
    area_b = (bx2 - bx1) * (by2 - by1)

    ltx = jnp.maximum(bx1, px1[None, :])
    lty = jnp.maximum(by1, py1[None, :])
    rbx = jnp.minimum(bx2, px2[None, :])
    rby = jnp.minimum(by2, py2[None, :])
    inter = jnp.clip(rbx - ltx, 0.0) * jnp.clip(rby - lty, 0.0)
    union = area_b + area_p[None, :] - inter
    ov = inter / union

    ofp = jnp.max(ov, axis=0)
    obj = jnp.argmax(ov, axis=0)

    oh = (obj[None, :] == lax.broadcasted_iota(jnp.int32, (_NOBJ, _P), 0))
    lab_col = labels_ref[0]
    lab = jnp.sum(jnp.where(oh, lab_col, 0), axis=0)
    lab = jnp.where(ofp < _THRESH, 0, lab)
    pos = lab != 0
    n_pos = jnp.sum(pos.astype(jnp.float32))

    gx1 = jnp.sum(jnp.where(oh, bx1, 0.0), axis=0)
    gy1 = jnp.sum(jnp.where(oh, by1, 0.0), axis=0)
    gx2 = jnp.sum(jnp.where(oh, bx2, 0.0), axis=0)
    gy2 = jnp.sum(jnp.where(oh, by2, 0.0), axis=0)

    gcx = (gx1 + gx2) * 0.5
    gcy = (gy1 + gy2) * 0.5
    gw = gx2 - gx1
    gh = gy2 - gy1
    t0 = (gcx - pcx) / (pw / 10.0)
    t1 = (gcy - pcy) / (ph / 10.0)
    t2 = jnp.log(gw / pw) * 5.0
    t3 = jnp.log(gh / ph) * 5.0

    l1 = (jnp.abs(locs_ref[0, 0, 0, :] - t0)
          + jnp.abs(locs_ref[1, 0, 0, :] - t1)
          + jnp.abs(locs_ref[2, 0, 0, :] - t2)
          + jnp.abs(locs_ref[3, 0, 0, :] - t3))
    loc_sum = jnp.sum(jnp.where(pos, l1, 0.0))

    s = scores_ref[:, 0, 0, :]
    m = jnp.max(s, axis=0)
    z = jnp.sum(jnp.exp(s - m[None, :]), axis=0)
    logz = m + jnp.log(z)
    s0 = s[0, :]

    onehot_pos = ((lab[None, :] ==
                   lax.broadcasted_iota(jnp.int32, (_C, _P), 0))
                  & (lab[None, :] != 0))
    ts_pos_sum = jnp.sum(jnp.where(onehot_pos, s, 0.0))
    logz_pos_sum = jnp.sum(jnp.where(pos, logz, 0.0))
    ce_pos_sum = logz_pos_sum - ts_pos_sum

    ce_neg = jnp.where(pos, 0.0, logz - s0)
    ce_ref[0, 0, :] = jnp.pad(ce_neg, (0, _PPAD - _P))

    lane = lax.broadcasted_iota(jnp.int32, (128,), 0)
    stats_ref[0, 0, :] = jnp.where(
        lane == 0, n_pos,
        jnp.where(lane == 1, ce_pos_sum,
                  jnp.where(lane == 2, loc_sum, 0.0)))


def _run_tc(scores_t, locs_t, boxes, labels3, priors_t):
    return pl.pallas_call(
        _tc_body,
        grid=(_B,),
        in_specs=[
            pl.BlockSpec((_C, 1, 1, _P), lambda b: (0, b, 0, 0)),
            pl.BlockSpec((4, 1, 1, _P), lambda b: (0, b, 0, 0)),
            pl.BlockSpec((1, _NOBJ, 4), lambda b: (b, 0, 0)),
            pl.BlockSpec((1, _NOBJ, 1), lambda b: (b, 0, 0)),
            pl.BlockSpec((4, _P), lambda b: (0, 0)),
        ],
        out_specs=[
            pl.BlockSpec((1, 1, _PPAD), lambda b: (b, 0, 0)),
            pl.BlockSpec((1, 1, 128), lambda b: (b, 0, 0)),
        ],
        out_shape=[
            jax.ShapeDtypeStruct((_B, 1, _PPAD), jnp.float32),
            jax.ShapeDtypeStruct((_B, 1, 128), jnp.float32),
        ],
        compiler_params=pltpu.CompilerParams(
            dimension_semantics=("arbitrary",)),
    )(scores_t, locs_t, boxes, labels3, priors_t)


@functools.cache
def _make_sc_hard():
    mesh = plsc.VectorSubcoreMesh(core_axis_name="c", subcore_axis_name="s")

    @functools.partial(
        pl.kernel,
        mesh=mesh,
        out_type=jax.ShapeDtypeStruct((_B, 16), jnp.float32),
        scratch_types=[
            pltpu.VMEM((_PPAD,), jnp.float32),
            pltpu.VMEM((128,), jnp.float32),
            pltpu.VMEM((16,), jnp.float32),
        ],
        compiler_params=pltpu.CompilerParams(needs_layout_passes=False),
    )
    def sc_hard(ce_hbm, stats_hbm, hard_hbm, ce_v, st_v, out_v):
        wid = lax.axis_index("s") * 2 + lax.axis_index("c")
        pltpu.sync_copy(ce_hbm.at[wid], ce_v)
        pltpu.sync_copy(stats_hbm.at[wid], st_v)

        npos = st_v[pl.ds(0, 16)][0]
        k_i = (_NEG_POS_RATIO * npos).astype(jnp.int32)
        k_v = jnp.broadcast_to(k_i, (16,))

        def outer(_, carry):
            lo_v, hi_v = carry
            mid_v = lo_v + lax.shift_right_logical(hi_v - lo_v + 1, 1)
            t_v = lax.bitcast_convert_type(mid_v, jnp.float32)

            def count_body(i, cnt):
                for u in range(8):
                    v = ce_v[pl.ds((i * 8 + u) * 16, 16)]
                    cnt = cnt + plsc.all_reduce_population_count(v >= t_v)
                return cnt

            cnt_v = lax.fori_loop(0, _NVEC // 8, count_body,
                                  jnp.zeros((16,), jnp.int32))
            ok = cnt_v >= k_v
            return (jnp.where(ok, mid_v, lo_v),
                    jnp.where(ok, hi_v, mid_v - 1))

        lo_v, hi_v = lax.fori_loop(
            0, 31, outer,
            (jnp.zeros((16,), jnp.int32),
             jnp.full((16,), _MAXF_BITS, jnp.int32)))
        t_v = lax.bitcast_convert_type(lo_v, jnp.float32)

        def sum_body(i, carry):
            s_acc, c_acc = carry
            for u in range(8):
                v = ce_v[pl.ds((i * 8 + u) * 16, 16)]
                gt = v > t_v
                s_acc = s_acc + jnp.where(gt, v, 0.0)
                c_acc = c_acc + plsc.all_reduce_population_count(gt)
            return (s_acc, c_acc)

        s_v, cgt_v = lax.fori_loop(
            0, _NVEC // 8, sum_body,
            (jnp.zeros((16,), jnp.float32), jnp.zeros((16,), jnp.int32)))

        sum_gt = s_v[0]
        for i in range(1, 16):
            sum_gt = sum_gt + s_v[i]
        cnt_gt = cgt_v[0]
        t_s = t_v[0]
        hard = sum_gt + (k_i - cnt_gt).astype(jnp.float32) * t_s

        out_v[...] = jnp.broadcast_to(hard, (16,))
        pltpu.sync_copy(out_v, hard_hbm.at[wid])

    return sc_hard


def _fin_body(stats_ref, hard_ref, out_ref):
    st = stats_ref[...]
    npos_tot = jnp.sum(st[:, 0])
    ce_pos_tot = jnp.sum(st[:, 1])
    loc_tot = jnp.sum(st[:, 2])
    hard_tot = jnp.sum(hard_ref[:, 0])
    loss = (hard_tot + ce_pos_tot) / npos_tot + loc_tot / (npos_tot * 4.0)
    out_ref[0, :] = jnp.broadcast_to(loss, (128,))


def _run_fin(stats, hard):
    return pl.pallas_call(
        _fin_body,
        out_shape=jax.ShapeDtypeStruct((1, 128), jnp.float32),
    )(stats, hard)


def kernel(predicted_locs, predicted_scores, boxes, labels, priors_cxcy):
    scores_t = jnp.transpose(predicted_scores, (2, 0, 1))
    scores_t = scores_t.reshape(_C, _B, 1, _P)
    locs_t = jnp.transpose(predicted_locs, (2, 0, 1))
    locs_t = locs_t.reshape(4, _B, 1, _P)
    priors_t = priors_cxcy.T
    labels3 = labels.reshape(_B, _NOBJ, 1).astype(jnp.int32)
    ce, stats = _run_tc(scores_t, locs_t, boxes, labels3, priors_t)
    ce = ce.reshape(_B, _PPAD)
    stats = stats.reshape(_B, 128)
    hard = _make_sc_hard()(ce, stats)
    fin = _run_fin(stats, hard)
    return fin[0, 0]


# speedup vs baseline: 1.5164x; 1.0177x over previous
import functools

import jax
import jax.numpy as jnp
from jax import lax
from jax.experimental import pallas as pl
from jax.experimental.pallas import tpu as pltpu
from jax.experimental.pallas import tpu_sc as plsc

_THRESH = 0.5
_NEG_POS_RATIO = 3.0
_B = 32
_P = 8732
_PPAD = 8960
_C = 81
_NOBJ = 8
_NVEC = _PPAD // 16
_MAXF_BITS = 0x7F7FFFFF


def _tc_body(scores_ref, locs_ref, boxes_ref, labels_ref, priors_ref,
             ce_ref, stats_ref):
    pcx = priors_ref[0, :]
    pcy = priors_ref[1, :]
    pw = priors_ref[2, :]
    ph = priors_ref[3, :]
    px1 = pcx - pw * 0.5
    py1 = pcy - ph * 0.5
    px2 = pcx + pw * 0.5
    py2 = pcy + ph * 0.5
    area_p = (px2 - px1) * (py2 - py1)

    bx = boxes_ref[0]
    bx1 = bx[:, 0:1]
    by1 = bx[:, 1:2]
    bx2 = bx[:, 2:3]
    by2 = bx[:, 3:4]
    area_b = (bx2 - bx1) * (by2 - by1)

    ltx = jnp.maximum(bx1, px1[None, :])
    lty = jnp.maximum(by1, py1[None, :])
    rbx = jnp.minimum(bx2, px2[None, :])
    rby = jnp.minimum(by2, py2[None, :])
    inter = jnp.clip(rbx - ltx, 0.0) * jnp.clip(rby - lty, 0.0)
    union = area_b + area_p[None, :] - inter
    ov = inter / union

    ofp = jnp.max(ov, axis=0)
    obj = jnp.argmax(ov, axis=0)

    oh = (obj[None, :] == lax.broadcasted_iota(jnp.int32, (_NOBJ, _P), 0))
    lab_col = labels_ref[0]
    lab = jnp.sum(jnp.where(oh, lab_col, 0), axis=0)
    lab = jnp.where(ofp < _THRESH, 0, lab)
    pos = lab != 0
    n_pos = jnp.sum(pos.astype(jnp.float32))

    gx1 = jnp.sum(jnp.where(oh, bx1, 0.0), axis=0)
    gy1 = jnp.sum(jnp.where(oh, by1, 0.0), axis=0)
    gx2 = jnp.sum(jnp.where(oh, bx2, 0.0), axis=0)
    gy2 = jnp.sum(jnp.where(oh, by2, 0.0), axis=0)

    gcx = (gx1 + gx2) * 0.5
    gcy = (gy1 + gy2) * 0.5
    gw = gx2 - gx1
    gh = gy2 - gy1
    t0 = (gcx - pcx) / (pw / 10.0)
    t1 = (gcy - pcy) / (ph / 10.0)
    t2 = jnp.log(gw / pw) * 5.0
    t3 = jnp.log(gh / ph) * 5.0

    l1 = (jnp.abs(locs_ref[0, 0, 0, :] - t0)
          + jnp.abs(locs_ref[1, 0, 0, :] - t1)
          + jnp.abs(locs_ref[2, 0, 0, :] - t2)
          + jnp.abs(locs_ref[3, 0, 0, :] - t3))
    loc_sum = jnp.sum(jnp.where(pos, l1, 0.0))

    s = scores_ref[:, 0, 0, :]
    m = jnp.max(s, axis=0)
    z = jnp.sum(jnp.exp(s - m[None, :]), axis=0)
    logz = m + jnp.log(z)
    s0 = s[0, :]

    onehot = (lab[None, :] ==
              lax.broadcasted_iota(jnp.int32, (_C, _P), 0))
    ts_all_sum = jnp.sum(jnp.where(onehot, s, 0.0))
    neg_s0_sum = jnp.sum(jnp.where(pos, 0.0, s0))
    logz_pos_sum = jnp.sum(jnp.where(pos, logz, 0.0))
    ce_pos_sum = logz_pos_sum - (ts_all_sum - neg_s0_sum)

    ce_neg = jnp.where(pos, 0.0, logz - s0)
    ce_ref[0, 0, :] = jnp.pad(ce_neg, (0, _PPAD - _P))

    lane = lax.broadcasted_iota(jnp.int32, (128,), 0)
    stats_ref[0, 0, :] = jnp.where(
        lane == 0, n_pos,
        jnp.where(lane == 1, ce_pos_sum,
                  jnp.where(lane == 2, loc_sum, 0.0)))


def _run_tc(scores_t, locs_t, boxes, labels3, priors_t):
    return pl.pallas_call(
        _tc_body,
        grid=(_B,),
        in_specs=[
            pl.BlockSpec((_C, 1, 1, _P), lambda b: (0, b, 0, 0)),
            pl.BlockSpec((4, 1, 1, _P), lambda b: (0, b, 0, 0)),
            pl.BlockSpec((1, _NOBJ, 4), lambda b: (b, 0, 0)),
            pl.BlockSpec((1, _NOBJ, 1), lambda b: (b, 0, 0)),
            pl.BlockSpec((4, _P), lambda b: (0, 0)),
        ],
        out_specs=[
            pl.BlockSpec((1, 1, _PPAD), lambda b: (b, 0, 0)),
            pl.BlockSpec((1, 1, 128), lambda b: (b, 0, 0)),
        ],
        out_shape=[
            jax.ShapeDtypeStruct((_B, 1, _PPAD), jnp.float32),
            jax.ShapeDtypeStruct((_B, 1, 128), jnp.float32),
        ],
        compiler_params=pltpu.CompilerParams(
            dimension_semantics=("arbitrary",)),
    )(scores_t, locs_t, boxes, labels3, priors_t)


@functools.cache
def _make_sc_hard():
    mesh = plsc.VectorSubcoreMesh(core_axis_name="c", subcore_axis_name="s")

    @functools.partial(
        pl.kernel,
        mesh=mesh,
        out_type=jax.ShapeDtypeStruct((_B, 16), jnp.float32),
        scratch_types=[
            pltpu.VMEM((_PPAD,), jnp.float32),
            pltpu.VMEM((128,), jnp.float32),
            pltpu.VMEM((16,), jnp.float32),
        ],
        compiler_params=pltpu.CompilerParams(needs_layout_passes=False),
    )
    def sc_hard(ce_hbm, stats_hbm, hard_hbm, ce_v, st_v, out_v):
        wid = lax.axis_index("s") * 2 + lax.axis_index("c")
        pltpu.sync_copy(ce_hbm.at[wid], ce_v)
        pltpu.sync_copy(stats_hbm.at[wid], st_v)

        npos = st_v[pl.ds(0, 16)][0]
        k_i = (_NEG_POS_RATIO * npos).astype(jnp.int32)
        k_v = jnp.broadcast_to(k_i, (16,))

        def outer(_, carry):
            lo_v, hi_v = carry
            mid_v = lo_v + lax.shift_right_logical(hi_v - lo_v + 1, 1)
            t_v = lax.bitcast_convert_type(mid_v, jnp.float32)

            def count_body(i, cnt):
                for u in range(16):
                    v = ce_v[pl.ds((i * 16 + u) * 16, 16)]
                    cnt = cnt + plsc.all_reduce_population_count(v >= t_v)
                return cnt

            cnt_v = lax.fori_loop(0, _NVEC // 16, count_body,
                                  jnp.zeros((16,), jnp.int32))
            ok = cnt_v >= k_v
            return (jnp.where(ok, mid_v, lo_v),
                    jnp.where(ok, hi_v, mid_v - 1))

        lo_v, hi_v = lax.fori_loop(
            0, 31, outer,
            (jnp.zeros((16,), jnp.int32),
             jnp.full((16,), _MAXF_BITS, jnp.int32)))
        t_v = lax.bitcast_convert_type(lo_v, jnp.float32)

        def sum_body(i, carry):
            s_acc, c_acc = carry
            for u in range(16):
                v = ce_v[pl.ds((i * 16 + u) * 16, 16)]
                gt = v > t_v
                s_acc = s_acc + jnp.where(gt, v, 0.0)
                c_acc = c_acc + plsc.all_reduce_population_count(gt)
            return (s_acc, c_acc)

        s_v, cgt_v = lax.fori_loop(
            0, _NVEC // 16, sum_body,
            (jnp.zeros((16,), jnp.float32), jnp.zeros((16,), jnp.int32)))

        sum_gt = s_v[0]
        for i in range(1, 16):
            sum_gt = sum_gt + s_v[i]
        cnt_gt = cgt_v[0]
        t_s = t_v[0]
        hard = sum_gt + (k_i - cnt_gt).astype(jnp.float32) * t_s

        out_v[...] = jnp.broadcast_to(hard, (16,))
        pltpu.sync_copy(out_v, hard_hbm.at[wid])

    return sc_hard


def _fin_body(stats_ref, hard_ref, out_ref):
    st = stats_ref[...]
    npos_tot = jnp.sum(st[:, 0])
    ce_pos_tot = jnp.sum(st[:, 1])
    loc_tot = jnp.sum(st[:, 2])
    hard_tot = jnp.sum(hard_ref[:, 0])
    loss = (hard_tot + ce_pos_tot) / npos_tot + loc_tot / (npos_tot * 4.0)
    out_ref[0, :] = jnp.broadcast_to(loss, (128,))


def _run_fin(stats, hard):
    return pl.pallas_call(
        _fin_body,
        out_shape=jax.ShapeDtypeStruct((1, 128), jnp.float32),
    )(stats, hard)


def kernel(predicted_locs, predicted_scores, boxes, labels, priors_cxcy):
    scores_t = jnp.transpose(predicted_scores, (2, 0, 1))
    scores_t = scores_t.reshape(_C, _B, 1, _P)
    locs_t = jnp.transpose(predicted_locs, (2, 0, 1))
    locs_t = locs_t.reshape(4, _B, 1, _P)
    priors_t = priors_cxcy.T
    labels3 = labels.reshape(_B, _NOBJ, 1).astype(jnp.int32)
    ce, stats = _run_tc(scores_t, locs_t, boxes, labels3, priors_t)
    ce = ce.reshape(_B, _PPAD)
    stats = stats.reshape(_B, 128)
    hard = _make_sc_hard()(ce, stats)
    fin = _run_fin(stats, hard)
    return fin[0, 0]
